# baseline (speedup 1.0000x reference)
import functools

import jax
import jax.numpy as jnp
from jax import lax
from jax.experimental import pallas as pl
from jax.experimental.pallas import tpu as pltpu
from jax.experimental.pallas import tpu_sc as plsc

N = 10000
E = 320000
D = 128
H = 128
G = 64
F = 8

NC = 2
NS = 16
NW = NC * NS
W = 125
CW = 8
NCH = 10
NWIN = CW * NCH
EPW = NWIN * W
EPAD = NW * EPW
NP = 10240
RPS = NP // NS
RING = 2

_sc_mesh = plsc.VectorSubcoreMesh(core_axis_name="c", subcore_axis_name="s")


@functools.partial(
    pl.kernel,
    out_type=jax.ShapeDtypeStruct((NC, NP, D), jnp.float32),
    mesh=_sc_mesh,
    scratch_types=[
        pltpu.VMEM((2, CW, W), jnp.int32),
        pltpu.VMEM((2, CW, W), jnp.int32),
        pltpu.VMEM((RING, W, D), jnp.float32),
        pltpu.VMEM_SHARED((NP, D), jnp.float32),
        pltpu.SemaphoreType.DMA,
        pltpu.SemaphoreType.DMA,
        pltpu.SemaphoreType.DMA,
        pltpu.SemaphoreType.DMA,
        pltpu.SemaphoreType.DMA,
    ],
)
def _sc_agg(h_hbm, edge_hbm, zeros_hbm, out_hbm,
            src_v, dst_v, rows_v, agg_sh, semi, sem0, sem1, semS0, semS1):
    c = lax.axis_index("c")
    s = lax.axis_index("s")
    wid = s * NC + c
    sems = (sem0, sem1)
    semSs = (semS0, semS1)
    row0 = wid * NWIN
    pltpu.sync_copy(zeros_hbm, agg_sh.at[pl.ds(s * RPS, RPS)])
    pltpu.sync_copy(edge_hbm.at[0, pl.ds(row0, CW)], src_v.at[0])
    pltpu.sync_copy(edge_hbm.at[1, pl.ds(row0, CW)], dst_v.at[0])
    plsc.subcore_barrier()
    for k in range(RING - 1):
        pltpu.async_copy(h_hbm.at[src_v.at[0, k]], rows_v.at[k], sems[k])

    @pl.loop(0, NCH)
    def _(ch):
        slot = lax.rem(ch, 2)
        nslot = 1 - slot

        @pl.when(ch + 1 < NCH)
        def _():
            pltpu.async_copy(edge_hbm.at[0, pl.ds(row0 + (ch + 1) * CW, CW)],
                             src_v.at[nslot], semi)
            pltpu.async_copy(edge_hbm.at[1, pl.ds(row0 + (ch + 1) * CW, CW)],
                             dst_v.at[nslot], semi)

        for k in range(CW):
            ka = k + RING - 1
            if k == 0:
                @pl.when(ch > 0)
                def _():
                    pltpu.make_async_copy(
                        rows_v.at[ka % RING],
                        agg_sh.at[dst_v.at[slot, k]], semSs[ka % RING]).wait()
            else:
                pltpu.make_async_copy(
                    rows_v.at[ka % RING],
                    agg_sh.at[dst_v.at[slot, k]], semSs[ka % RING]).wait()
            if ka < CW:
                pltpu.async_copy(h_hbm.at[src_v.at[slot, ka]],
                                 rows_v.at[ka % RING], sems[ka % RING])
            else:
                @pl.when(ch + 1 < NCH)
                def _():
                    if ka == CW:
                        pltpu.make_async_copy(
                            edge_hbm.at[0, pl.ds(row0, CW)], src_v.at[nslot],
                            semi).wait()
                        pltpu.make_async_copy(
                            edge_hbm.at[1, pl.ds(row0, CW)], dst_v.at[nslot],
                            semi).wait()
                    pltpu.async_copy(h_hbm.at[src_v.at[nslot, ka - CW]],
                                     rows_v.at[ka % RING], sems[ka % RING])
            pltpu.make_async_copy(h_hbm.at[src_v.at[slot, k]],
                                  rows_v.at[k % RING], sems[k % RING]).wait()
            pltpu.async_copy(rows_v.at[k % RING],
                             agg_sh.at[dst_v.at[slot, k]],
                             semSs[k % RING], add=True)

    pltpu.make_async_copy(rows_v.at[(NWIN - 1) % RING],
                          agg_sh.at[dst_v.at[0, 0]],
                          semSs[(NWIN - 1) % RING]).wait()
    plsc.subcore_barrier()
    pltpu.sync_copy(agg_sh.at[pl.ds(s * RPS, RPS)],
                    out_hbm.at[c, pl.ds(s * RPS, RPS)])


def _tc_layer_body(h_ref, a_ref, w1_ref, b1_ref, g1_ref, bt1_ref,
                   w2_ref, b2_ref, h_out_ref):
    a = a_ref[...]
    hin = h_ref[...] + a[0, :N] + a[1, :N]
    z = jnp.dot(hin, w1_ref[...], preferred_element_type=jnp.float32) + b1_ref[...]
    mean = jnp.mean(z, axis=0, keepdims=True)
    zc = z - mean
    var = jnp.mean(zc * zc, axis=0, keepdims=True)
    zn = zc * lax.rsqrt(var + 1e-5) * g1_ref[...] + bt1_ref[...]
    zn = jnp.maximum(zn, 0.0)
    h2 = jnp.dot(zn, w2_ref[...], preferred_element_type=jnp.float32) + b2_ref[...]
    h_out_ref[...] = jnp.maximum(h2, 0.0)


_tc_layer = pl.pallas_call(
    _tc_layer_body,
    out_shape=jax.ShapeDtypeStruct((N, H), jnp.float32),
)


def _tc_pool_body(h_ref, batch_ref, pooled_ref):
    gids = lax.broadcasted_iota(jnp.int32, (G, N), 0)
    mask = (batch_ref[...] == gids).astype(jnp.float32)
    psum = jnp.dot(mask, h_ref[...], preferred_element_type=jnp.float32)
    cnt = jnp.maximum(jnp.sum(mask, axis=1, keepdims=True), 1.0)
    pooled_ref[...] = psum / cnt


_tc_pool = pl.pallas_call(
    _tc_pool_body,
    out_shape=jax.ShapeDtypeStruct((G, H), jnp.float32),
)


def _tc_head_body(hcat_ref, w1_ref, b1_ref, w2_ref, b2_ref, out_ref):
    h = jnp.dot(hcat_ref[...], w1_ref[...], preferred_element_type=jnp.float32) + b1_ref[...]
    h = jnp.maximum(h, 0.0)
    out_ref[...] = jnp.dot(h, w2_ref[...], preferred_element_type=jnp.float32) + b2_ref[...]


_tc_head = pl.pallas_call(
    _tc_head_body,
    out_shape=jax.ShapeDtypeStruct((G, F), jnp.float32),
)


def kernel(x, edge_index, batch, params):
    npad = EPAD - E
    if npad:
        pad = jnp.concatenate(
            [jnp.zeros((1, npad), jnp.int32), jnp.full((1, npad), N, jnp.int32)])
        edge_index = jnp.concatenate([edge_index, pad], axis=1)
    edge3 = edge_index.reshape(2, NW * NWIN, W)
    zeros = jnp.zeros((RPS, D), jnp.float32)
    batch2 = batch.reshape(1, N)
    h = x
    pooled = []
    for name in ("conv1", "conv2", "conv3", "conv4", "conv5"):
        p = params[name]
        agg = _sc_agg(h, edge3, zeros)
        h = _tc_layer(
            h, agg,
            p["W1"], p["b1"].reshape(1, H), p["g1"].reshape(1, H),
            p["beta1"].reshape(1, H), p["W2"], p["b2"].reshape(1, H))
        pooled.append(_tc_pool(h, batch2))
    hcat = jnp.concatenate(pooled, axis=1)
    out = _tc_head(hcat, params["lin1_W"], params["lin1_b"].reshape(1, 5 * H),
                   params["lin2_W"], params["lin2_b"].reshape(1, F))
    return out.reshape(-1)


# speedup vs baseline: 1.0087x; 1.0087x over previous
import functools

import jax
import jax.numpy as jnp
from jax import lax
from jax.experimental import pallas as pl
from jax.experimental.pallas import tpu as pltpu
from jax.experimental.pallas import tpu_sc as plsc

N = 10000
E = 320000
D = 128
H = 128
G = 64
F = 8

NC = 2
NS = 16
NW = NC * NS
W = 125
CW = 8
NCH = 10
NWIN = CW * NCH
EPW = NWIN * W
EPAD = NW * EPW
NP = 10240
RPS = NP // NS
RING = 2

_sc_mesh = plsc.VectorSubcoreMesh(core_axis_name="c", subcore_axis_name="s")


@functools.partial(
    pl.kernel,
    out_type=jax.ShapeDtypeStruct((NC, NP, D), jnp.float32),
    mesh=_sc_mesh,
    scratch_types=[
        pltpu.VMEM((2, CW, W), jnp.int32),
        pltpu.VMEM((2, CW, W), jnp.int32),
        pltpu.VMEM((RING, W, D), jnp.float32),
        pltpu.VMEM_SHARED((NP, D), jnp.float32),
        pltpu.SemaphoreType.DMA,
        pltpu.SemaphoreType.DMA,
        pltpu.SemaphoreType.DMA,
        pltpu.SemaphoreType.DMA,
        pltpu.SemaphoreType.DMA,
    ],
)
def _sc_agg(h_hbm, edge_hbm, zeros_hbm, out_hbm,
            src_v, dst_v, rows_v, agg_sh, semi, sem0, sem1, semS0, semS1):
    c = lax.axis_index("c")
    s = lax.axis_index("s")
    wid = s * NC + c
    sems = (sem0, sem1)
    semSs = (semS0, semS1)
    row0 = wid * NWIN
    pltpu.sync_copy(zeros_hbm, agg_sh.at[pl.ds(s * RPS, RPS)])
    pltpu.sync_copy(edge_hbm.at[0, pl.ds(row0, CW)], src_v.at[0])
    pltpu.sync_copy(edge_hbm.at[1, pl.ds(row0, CW)], dst_v.at[0])
    plsc.subcore_barrier()
    for k in range(RING - 1):
        pltpu.async_copy(h_hbm.at[src_v.at[0, k]], rows_v.at[k], sems[k])

    @pl.loop(0, NCH)
    def _(ch):
        slot = lax.rem(ch, 2)
        nslot = 1 - slot

        @pl.when(ch + 1 < NCH)
        def _():
            pltpu.async_copy(edge_hbm.at[0, pl.ds(row0 + (ch + 1) * CW, CW)],
                             src_v.at[nslot], semi)
            pltpu.async_copy(edge_hbm.at[1, pl.ds(row0 + (ch + 1) * CW, CW)],
                             dst_v.at[nslot], semi)

        for k in range(CW):
            ka = k + RING - 1
            if k == 0:
                @pl.when(ch > 0)
                def _():
                    pltpu.make_async_copy(
                        rows_v.at[ka % RING],
                        agg_sh.at[dst_v.at[slot, k]], semSs[ka % RING]).wait()
            else:
                pltpu.make_async_copy(
                    rows_v.at[ka % RING],
                    agg_sh.at[dst_v.at[slot, k]], semSs[ka % RING]).wait()
            if ka < CW:
                pltpu.async_copy(h_hbm.at[src_v.at[slot, ka]],
                                 rows_v.at[ka % RING], sems[ka % RING])
            else:
                @pl.when(ch + 1 < NCH)
                def _():
                    if ka == CW:
                        pltpu.make_async_copy(
                            edge_hbm.at[0, pl.ds(row0, CW)], src_v.at[nslot],
                            semi).wait()
                        pltpu.make_async_copy(
                            edge_hbm.at[1, pl.ds(row0, CW)], dst_v.at[nslot],
                            semi).wait()
                    pltpu.async_copy(h_hbm.at[src_v.at[nslot, ka - CW]],
                                     rows_v.at[ka % RING], sems[ka % RING])
            pltpu.make_async_copy(h_hbm.at[src_v.at[slot, k]],
                                  rows_v.at[k % RING], sems[k % RING]).wait()
            pltpu.async_copy(rows_v.at[k % RING],
                             agg_sh.at[dst_v.at[slot, k]],
                             semSs[k % RING], add=True)

    pltpu.make_async_copy(rows_v.at[(NWIN - 1) % RING],
                          agg_sh.at[dst_v.at[0, 0]],
                          semSs[(NWIN - 1) % RING]).wait()
    plsc.subcore_barrier()
    pltpu.sync_copy(agg_sh.at[pl.ds(s * RPS, RPS)],
                    out_hbm.at[c, pl.ds(s * RPS, RPS)])


def _tc_layer_body(h_ref, a_ref, w1_ref, b1_ref, g1_ref, bt1_ref,
                   w2_ref, b2_ref, batch_ref, h_out_ref, pooled_ref):
    a = a_ref[...]
    hin = h_ref[...] + a[0, :N] + a[1, :N]
    z = jnp.dot(hin, w1_ref[...], preferred_element_type=jnp.float32) + b1_ref[...]
    mean = jnp.mean(z, axis=0, keepdims=True)
    zc = z - mean
    var = jnp.mean(zc * zc, axis=0, keepdims=True)
    zn = zc * lax.rsqrt(var + 1e-5) * g1_ref[...] + bt1_ref[...]
    zn = jnp.maximum(zn, 0.0)
    h2 = jnp.dot(zn, w2_ref[...], preferred_element_type=jnp.float32) + b2_ref[...]
    h2 = jnp.maximum(h2, 0.0)
    h_out_ref[...] = h2
    gids = lax.broadcasted_iota(jnp.int32, (G, N), 0)
    mask = (batch_ref[...] == gids).astype(jnp.float32)
    psum = jnp.dot(mask, h2, preferred_element_type=jnp.float32)
    cnt = jnp.maximum(jnp.sum(mask, axis=1, keepdims=True), 1.0)
    pooled_ref[...] = psum / cnt


_tc_layer = pl.pallas_call(
    _tc_layer_body,
    out_shape=(
        jax.ShapeDtypeStruct((N, H), jnp.float32),
        jax.ShapeDtypeStruct((G, H), jnp.float32),
    ),
)


def _tc_head_body(hcat_ref, w1_ref, b1_ref, w2_ref, b2_ref, out_ref):
    h = jnp.dot(hcat_ref[...], w1_ref[...], preferred_element_type=jnp.float32) + b1_ref[...]
    h = jnp.maximum(h, 0.0)
    out_ref[...] = jnp.dot(h, w2_ref[...], preferred_element_type=jnp.float32) + b2_ref[...]


_tc_head = pl.pallas_call(
    _tc_head_body,
    out_shape=jax.ShapeDtypeStruct((G, F), jnp.float32),
)


def kernel(x, edge_index, batch, params):
    npad = EPAD - E
    if npad:
        pad = jnp.concatenate(
            [jnp.zeros((1, npad), jnp.int32), jnp.full((1, npad), N, jnp.int32)])
        edge_index = jnp.concatenate([edge_index, pad], axis=1)
    edge3 = edge_index.reshape(2, NW * NWIN, W)
    zeros = jnp.zeros((RPS, D), jnp.float32)
    batch2 = batch.reshape(1, N)
    h = x
    pooled = []
    for name in ("conv1", "conv2", "conv3", "conv4", "conv5"):
        p = params[name]
        agg = _sc_agg(h, edge3, zeros)
        h, pool = _tc_layer(
            h, agg,
            p["W1"], p["b1"].reshape(1, H), p["g1"].reshape(1, H),
            p["beta1"].reshape(1, H), p["W2"], p["b2"].reshape(1, H),
            batch2)
        pooled.append(pool)
    hcat = jnp.concatenate(pooled, axis=1)
    out = _tc_head(hcat, params["lin1_W"], params["lin1_b"].reshape(1, 5 * H),
                   params["lin2_W"], params["lin2_b"].reshape(1, F))
    return out.reshape(-1)
